# baseline (speedup 1.0000x reference)
import functools

import jax
import jax.numpy as jnp
from jax.experimental import pallas as pl
from jax.experimental.pallas import tpu as pltpu

_ABSORB = 0.5
_K = 8
_EPS = 1e-5


def _absorb(f, w1_ref, b1_ref, g_ref, be_ref, w2_ref, fo_scr, sim_scr, dst):
    bs, n, d = f.shape
    ff = f.reshape(bs * n, d)
    p = jnp.dot(ff, w1_ref[:d], preferred_element_type=jnp.float32)
    q = jnp.dot(ff, w1_ref[d:], preferred_element_type=jnp.float32)

    ex = jnp.mean(p, axis=0)
    ey = jnp.mean(q, axis=0)
    ex2 = jnp.mean(p * p, axis=0)
    ey2 = jnp.mean(q * q, axis=0)
    pb = jnp.mean(p.reshape(bs, n, d), axis=1)
    qb = jnp.mean(q.reshape(bs, n, d), axis=1)
    exy = jnp.mean(pb * qb, axis=0)
    mu_xy = ex + ey
    var = ex2 + ey2 + 2.0 * exy - mu_xy * mu_xy

    alpha = g_ref[0] * jax.lax.rsqrt(var + _EPS)
    beta = be_ref[0] - alpha * mu_xy
    w2 = w2_ref[0]

    at = alpha[None, None, :] * p.reshape(bs, n, d) + beta[None, None, :]
    bt = alpha[None, None, :] * q.reshape(bs, n, d)

    row_i = jax.lax.broadcasted_iota(jnp.int32, (n, n), 0)
    col_j = jax.lax.broadcasted_iota(jnp.int32, (n, n), 1)

    ti = 16
    for b in range(bs):
        for it in range(n // ti):
            t = at[b, it * ti:(it + 1) * ti][:, None, :] + bt[b][None, :, :]
            t = jnp.maximum(t, 0.0)
            sim_scr[it * ti:(it + 1) * ti, :] = jnp.sum(t * w2[None, None, :],
                                                        axis=-1)
        work = jnp.where(row_i == col_j, -1e9, sim_scr[...])
        mask = jnp.zeros((n, n), jnp.float32)
        for _ in range(_K):
            mx = jnp.max(work, axis=1, keepdims=True)
            eq = work == mx
            selj = jnp.min(jnp.where(eq, col_j, n), axis=1, keepdims=True)
            onehot = col_j == selj
            mask = mask + onehot.astype(jnp.float32)
            work = jnp.where(onehot, -jnp.inf, work)
        nei = jnp.dot(mask, f[b], preferred_element_type=jnp.float32) * (1.0 / _K)
        fo_scr[dst, b] = ((1.0 - _ABSORB) * f[b]
                          + _ABSORB * nei).astype(jnp.bfloat16)


def _gm_kernel(f1_ref, f2_ref, sw1_ref, sg_ref, sbe_ref, sw2_ref,
               cw1_ref, cg_ref, cbe_ref, cw2c_ref, cb2_ref,
               out_ref, fo_scr, h_scr, stats_scr, sim_scr, *, nt, ti):
    s = pl.program_id(0)
    bs, n, d = f1_ref.shape
    m_rows = float(bs * n * n)
    p0_end = 2 + bs * nt

    @pl.when(s == 0)
    def _():
        _absorb(f1_ref[...], sw1_ref, None, sg_ref, sbe_ref, sw2_ref,
                fo_scr, sim_scr, 0)

    @pl.when(s == 1)
    def _():
        _absorb(f2_ref[...], sw1_ref, None, sg_ref, sbe_ref, sw2_ref,
                fo_scr, sim_scr, 1)

    @pl.when(jnp.logical_and(s >= 2, s < p0_end))
    def _():
        idx = s - 2
        b = idx // nt
        it = idx % nt
        f1t = fo_scr[0, b, pl.ds(it * ti, ti)]
        f2b = fo_scr[1, b]
        diff = jnp.abs(f1t[:, None, :] - f2b[None, :, :])
        h = jnp.dot(diff.reshape(ti * n, d), cw1_ref[...].astype(jnp.bfloat16),
                    preferred_element_type=jnp.float32)

        @pl.when(s == 2)
        def _():
            stats_scr[...] = jnp.zeros_like(stats_scr)

        stats_scr[0:1, :] += jnp.sum(h, axis=0)[None, :]
        stats_scr[1:2, :] += jnp.sum(h * h, axis=0)[None, :]
        h_scr[b, pl.ds(it * ti, ti)] = h.reshape(ti, n, d).astype(jnp.bfloat16)

    @pl.when(s >= p0_end)
    def _():
        idx = s - p0_end
        b = idx // nt
        it = idx % nt
        mean = stats_scr[0] * (1.0 / m_rows)
        var = stats_scr[1] * (1.0 / m_rows) - mean * mean
        alpha = (cg_ref[0] * jax.lax.rsqrt(var + _EPS)).astype(jnp.bfloat16)
        beta = (cbe_ref[0] - cg_ref[0] * jax.lax.rsqrt(var + _EPS) * mean
                ).astype(jnp.bfloat16)
        h = h_scr[b, pl.ds(it * ti, ti)]
        t = jnp.maximum(alpha[None, None, :] * h + beta[None, None, :],
                        jnp.bfloat16(0.0))
        sc = jnp.dot(
            t.reshape(ti * n, d), cw2c_ref[...].astype(jnp.bfloat16),
            preferred_element_type=jnp.float32).reshape(ti, n) + cb2_ref[0, 0]
        out_ref[0, 0] = jax.nn.sigmoid(sc)


def kernel(feat1, feat2, sim_w1, sim_b1, sim_g1, sim_be1, sim_w2, sim_b2,
           cls_w1, cls_b1, cls_g1, cls_be1, cls_w2, cls_b2):
    bs, n, d = feat1.shape
    ti = 64
    nt = n // ti
    p0_end = 2 + bs * nt
    steps = 2 + 2 * bs * nt
    row = lambda v: v.reshape(1, -1)
    b2_full = jnp.broadcast_to(cls_b2.reshape(1, 1), (1, d))

    const = lambda shape: pl.BlockSpec(shape, lambda s: (0,) * len(shape))

    def out_map(s):
        p = jnp.where(s >= p0_end, 1, 0)
        idx = jnp.maximum(s - p0_end, 0)
        return (p, idx // nt, idx % nt, 0)

    score = pl.pallas_call(
        functools.partial(_gm_kernel, nt=nt, ti=ti),
        grid=(steps,),
        in_specs=[
            const((bs, n, d)),
            const((bs, n, d)),
            const((2 * d, d)),
            const((1, d)),
            const((1, d)),
            const((1, d)),
            const((d, d)),
            const((1, d)),
            const((1, d)),
            const((d, 1)),
            const((1, d)),
        ],
        out_specs=pl.BlockSpec((1, 1, ti, n), out_map),
        out_shape=jax.ShapeDtypeStruct((2, bs, n, n), jnp.float32),
        scratch_shapes=[
            pltpu.VMEM((2, bs, n, d), jnp.bfloat16),
            pltpu.VMEM((bs, n, n, d), jnp.bfloat16),
            pltpu.VMEM((8, d), jnp.float32),
            pltpu.VMEM((n, n), jnp.float32),
        ],
    )(feat1, feat2, sim_w1, row(sim_g1), row(sim_be1), row(sim_w2),
      cls_w1, row(cls_g1), row(cls_be1), cls_w2, b2_full)

    return score[1]
